# Initial kernel scaffold; baseline (speedup 1.0000x reference)
import jax
import jax.numpy as jnp
from jax.experimental import pallas as pl


def kernel(pred, target):
    raise NotImplementedError("write your pallas kernel here")


# speedup vs baseline: 1.7669x; 1.7669x over previous
import functools

import jax
import jax.numpy as jnp
from jax.experimental import pallas as pl
from jax.experimental.pallas import tpu as pltpu

NBINS = 32
SMOOTH_NR = 1e-07
SMOOTH_DR = 1e-07


def _mi_kernel(
    pred_ref, targ_ref, out_ref, pab_acc, pa_acc, pb_acc, *, n_blocks, n_total, batch
):
    b = pl.program_id(0)
    m = pl.program_id(1)

    preterm = 0.5 * NBINS * NBINS
    centers = (
        jax.lax.broadcasted_iota(jnp.int32, (NBINS, 1), 0).astype(jnp.float32) + 0.5
    ) / NBINS

    @pl.when(m == 0)
    def _init():
        pab_acc[...] = jnp.zeros_like(pab_acc)
        pa_acc[...] = jnp.zeros_like(pa_acc)
        pb_acc[...] = jnp.zeros_like(pb_acc)

    x = pred_ref[0]
    y = targ_ref[0]

    da = x - centers
    wa = jnp.exp(-preterm * da * da)
    wa = wa / jnp.sum(wa, axis=0, keepdims=True)

    db = y - centers
    wb = jnp.exp(-preterm * db * db)
    wb = wb / jnp.sum(wb, axis=0, keepdims=True)

    pa_acc[...] += jnp.sum(wa, axis=1, keepdims=True)
    pb_acc[...] += jnp.sum(wb, axis=1, keepdims=True)
    pab_acc[...] += jax.lax.dot_general(
        wa, wb, (((1,), (1,)), ((), ())), preferred_element_type=jnp.float32
    )

    @pl.when(m == n_blocks - 1)
    def _epilogue():
        inv_n = 1.0 / n_total
        pab = pab_acc[...] * inv_n
        pa = pa_acc[...] * inv_n
        pb = jnp.transpose(pb_acc[...] * inv_n)
        papb = pa * pb
        mi = jnp.sum(
            pab * jnp.log((pab + SMOOTH_NR) / (papb + SMOOTH_DR) + SMOOTH_DR),
            keepdims=True,
        )

        @pl.when(b == 0)
        def _first():
            out_ref[...] = jnp.zeros_like(out_ref)

        out_ref[...] += -mi / batch


@functools.partial(jax.jit, static_argnames=())
def kernel(pred, target):
    B = pred.shape[0]
    n_total = 1
    for s in pred.shape[1:]:
        n_total *= s
    K = 16384
    M = n_total // K
    pf = pred.reshape(B * M, 1, K)
    tf = target.reshape(B * M, 1, K)

    out = pl.pallas_call(
        functools.partial(_mi_kernel, n_blocks=M, n_total=n_total, batch=B),
        grid=(B, M),
        in_specs=[
            pl.BlockSpec((1, 1, K), lambda b, m: (b * M + m, 0, 0)),
            pl.BlockSpec((1, 1, K), lambda b, m: (b * M + m, 0, 0)),
        ],
        out_specs=pl.BlockSpec((1, 1), lambda b, m: (0, 0)),
        out_shape=jax.ShapeDtypeStruct((1, 1), jnp.float32),
        scratch_shapes=[
            pltpu.VMEM((NBINS, NBINS), jnp.float32),
            pltpu.VMEM((NBINS, 1), jnp.float32),
            pltpu.VMEM((NBINS, 1), jnp.float32),
        ],
    )(pf, tf)
    return out.reshape(())


# speedup vs baseline: 1.8782x; 1.0630x over previous
import functools

import jax
import jax.numpy as jnp
from jax.experimental import pallas as pl
from jax.experimental.pallas import tpu as pltpu

NBINS = 32
SMOOTH_NR = 1e-07
SMOOTH_DR = 1e-07


def _mi_kernel(
    pred_ref, targ_ref, out_ref, pab_acc, pa_acc, pb_acc, *, n_blocks, n_total, batch
):
    b = pl.program_id(0)
    m = pl.program_id(1)

    c2 = -0.5 * NBINS * NBINS * 1.4426950408889634
    centers = (
        jax.lax.broadcasted_iota(jnp.int32, (NBINS, 1), 0).astype(jnp.float32) + 0.5
    ) / NBINS

    @pl.when(m == 0)
    def _init():
        pab_acc[...] = jnp.zeros_like(pab_acc)
        pa_acc[...] = jnp.zeros_like(pa_acc)
        pb_acc[...] = jnp.zeros_like(pb_acc)

    x = pred_ref[0]
    y = targ_ref[0]

    da = x - centers
    wa = jnp.exp2(c2 * (da * da))
    wa = wa / jnp.sum(wa, axis=0, keepdims=True)

    db = y - centers
    wb = jnp.exp2(c2 * (db * db))
    wb = wb / jnp.sum(wb, axis=0, keepdims=True)

    pa_acc[...] += jnp.sum(wa, axis=1, keepdims=True)
    pb_acc[...] += jnp.sum(wb, axis=1, keepdims=True)
    pab_acc[...] += jax.lax.dot_general(
        wa, wb, (((1,), (1,)), ((), ())), preferred_element_type=jnp.float32
    )

    @pl.when(m == n_blocks - 1)
    def _epilogue():
        inv_n = 1.0 / n_total
        pab = pab_acc[...] * inv_n
        pa = pa_acc[...] * inv_n
        pb = jnp.transpose(pb_acc[...] * inv_n)
        papb = pa * pb
        mi = jnp.sum(
            pab * jnp.log((pab + SMOOTH_NR) / (papb + SMOOTH_DR) + SMOOTH_DR),
            keepdims=True,
        )

        @pl.when(b == 0)
        def _first():
            out_ref[...] = jnp.zeros_like(out_ref)

        out_ref[...] += -mi / batch


@functools.partial(jax.jit, static_argnames=())
def kernel(pred, target):
    B = pred.shape[0]
    n_total = 1
    for s in pred.shape[1:]:
        n_total *= s
    K = 16384
    M = n_total // K
    pf = pred.reshape(B * M, 1, K)
    tf = target.reshape(B * M, 1, K)

    out = pl.pallas_call(
        functools.partial(_mi_kernel, n_blocks=M, n_total=n_total, batch=B),
        grid=(B, M),
        in_specs=[
            pl.BlockSpec((1, 1, K), lambda b, m: (b * M + m, 0, 0)),
            pl.BlockSpec((1, 1, K), lambda b, m: (b * M + m, 0, 0)),
        ],
        out_specs=pl.BlockSpec((1, 1), lambda b, m: (0, 0)),
        out_shape=jax.ShapeDtypeStruct((1, 1), jnp.float32),
        scratch_shapes=[
            pltpu.VMEM((NBINS, NBINS), jnp.float32),
            pltpu.VMEM((NBINS, 1), jnp.float32),
            pltpu.VMEM((NBINS, 1), jnp.float32),
        ],
    )(pf, tf)
    return out.reshape(())


# speedup vs baseline: 2.2244x; 1.1843x over previous
import functools

import jax
import jax.numpy as jnp
from jax.experimental import pallas as pl
from jax.experimental.pallas import tpu as pltpu

NBINS = 32
SMOOTH_NR = 1e-07
SMOOTH_DR = 1e-07


def _mi_kernel(
    pred_ref,
    targ_ref,
    out_ref,
    pab_acc,
    pa_acc,
    pb_acc,
    *,
    n_groups,
    group,
    n_total,
    batch,
):
    b = pl.program_id(0)
    m = pl.program_id(1)

    c2 = -0.5 * NBINS * NBINS * 1.4426950408889634
    centers = (
        jax.lax.broadcasted_iota(jnp.int32, (NBINS, 1), 0).astype(jnp.float32) + 0.5
    ) / NBINS

    @pl.when(m == 0)
    def _init():
        pab_acc[...] = jnp.zeros_like(pab_acc)
        pa_acc[...] = jnp.zeros_like(pa_acc)
        pb_acc[...] = jnp.zeros_like(pb_acc)

    for g in range(group):
        x = pred_ref[0, g : g + 1, :]
        y = targ_ref[0, g : g + 1, :]

        da = x - centers
        wa = jnp.exp2(c2 * (da * da))
        wa = wa / jnp.sum(wa, axis=0, keepdims=True)

        db = y - centers
        wb = jnp.exp2(c2 * (db * db))
        wb = wb / jnp.sum(wb, axis=0, keepdims=True)

        pa_acc[...] += jnp.sum(wa, axis=1, keepdims=True)
        pb_acc[...] += jnp.sum(wb, axis=1, keepdims=True)
        pab_acc[...] += jax.lax.dot_general(
            wa, wb, (((1,), (1,)), ((), ())), preferred_element_type=jnp.float32
        )

    @pl.when(m == n_groups - 1)
    def _epilogue():
        inv_n = 1.0 / n_total
        pab = pab_acc[...] * inv_n
        pa = pa_acc[...] * inv_n
        pb = jnp.transpose(pb_acc[...] * inv_n)
        papb = pa * pb
        mi = jnp.sum(
            pab * jnp.log((pab + SMOOTH_NR) / (papb + SMOOTH_DR) + SMOOTH_DR),
            keepdims=True,
        )

        @pl.when(b == 0)
        def _first():
            out_ref[...] = jnp.zeros_like(out_ref)

        out_ref[...] += -mi / batch


@functools.partial(jax.jit, static_argnames=())
def kernel(pred, target):
    B = pred.shape[0]
    n_total = 1
    for s in pred.shape[1:]:
        n_total *= s
    K = 16384
    G = 2
    M = n_total // K
    MG = M // G
    pf = pred.reshape(B * MG, G, K)
    tf = target.reshape(B * MG, G, K)

    out = pl.pallas_call(
        functools.partial(
            _mi_kernel, n_groups=MG, group=G, n_total=n_total, batch=B
        ),
        grid=(B, MG),
        in_specs=[
            pl.BlockSpec((1, G, K), lambda b, m: (b * MG + m, 0, 0)),
            pl.BlockSpec((1, G, K), lambda b, m: (b * MG + m, 0, 0)),
        ],
        out_specs=pl.BlockSpec((1, 1), lambda b, m: (0, 0)),
        out_shape=jax.ShapeDtypeStruct((1, 1), jnp.float32),
        scratch_shapes=[
            pltpu.VMEM((NBINS, NBINS), jnp.float32),
            pltpu.VMEM((NBINS, 1), jnp.float32),
            pltpu.VMEM((NBINS, 1), jnp.float32),
        ],
    )(pf, tf)
    return out.reshape(())
